# baseline (speedup 1.0000x reference)
import functools

import jax
import jax.numpy as jnp
from jax import lax
from jax.experimental import pallas as pl
from jax.experimental.pallas import tpu as pltpu
from jax.experimental.pallas import tpu_sc as plsc

N_NODES = 10000
N_EDGES = 320000
HID = 128
NUM_EMB = 125
NUM_RBF = 16
CUTOFF = 12.0
VNODE_Z = 124
GAMMA = (CUTOFF / NUM_RBF) ** -2
RBF_STEP = CUTOFF / (NUM_RBF - 1)

NC = 2
NS = 16
NW = NC * NS
EPW = N_EDGES // NW
N_PAD = 10240
ROWS_PER_TILE = N_PAD // NS

C1 = 400
C2 = 40
ZR = 128
QLUT = 8192
DELTA = 0.002


_HCH = 80
_NHCH = N_NODES // _HCH


def _sc1_body(src_hbm, dst_hbm, pos_hbm, an_hbm, emb_hbm, rbf_hbm, h_hbm,
              pos_v, src_all, dst_all, rb0, rb1, an_v, h_v,
              hsem, w0, w1):
    c = lax.axis_index("c")
    s = lax.axis_index("s")
    wid = c * NS + s
    base0 = wid * EPW
    pltpu.sync_copy(pos_hbm, pos_v)

    nch = 3 + (wid < _NHCH - 3 * NW).astype(jnp.int32)

    def hbody(q, carry):
        base = (wid + q * NW) * _HCH
        pltpu.sync_copy(an_hbm.at[pl.ds(base, _HCH)], an_v)
        pltpu.async_copy(emb_hbm.at[an_v], h_v, hsem).wait()
        pltpu.sync_copy(h_v, h_hbm.at[pl.ds(base, _HCH)])
        return carry

    lax.fori_loop(0, nch, hbody, 0)

    pltpu.sync_copy(src_hbm.at[pl.ds(base0, EPW)], src_all)
    pltpu.sync_copy(dst_hbm.at[pl.ds(base0, EPW)], dst_all)

    rbuf = (rb0, rb1)
    wsem = (w0, w1)

    def compute_chunk(j, b):
        def vec_body(k, carry2):
            loc = pl.multiple_of(j * C1 + k * 16, 16)
            si = src_all[pl.ds(loc, 16)] * 3
            di = dst_all[pl.ds(loc, 16)] * 3
            sx = plsc.load_gather(pos_v, [si])
            sy = plsc.load_gather(pos_v, [si + 1])
            sz = plsc.load_gather(pos_v, [si + 2])
            dx = plsc.load_gather(pos_v, [di]) - sx
            dy = plsc.load_gather(pos_v, [di + 1]) - sy
            dz = plsc.load_gather(pos_v, [di + 2]) - sz
            x = dx * dx + dy * dy + dz * dz + 1e-12
            ii = plsc.bitcast(x, jnp.int32)
            ii = 0x5F3759DF - lax.shift_right_logical(ii, 1)
            y = plsc.bitcast(ii, jnp.float32)
            for _ in range(4):
                y = y * (1.5 - 0.5 * x * y * y)
            d = x * y
            qf = d * (1.0 / DELTA) + 0.5
            qi = jnp.minimum(qf.astype(jnp.int32), QLUT - 1)
            rbuf[b][pl.ds(pl.multiple_of(k * 16, 16), 16)] = qi
            return carry2

        lax.fori_loop(0, C1 // 16, vec_body, 0)

    def wstart(j, b):
        pltpu.async_copy(rbuf[b], rbf_hbm.at[pl.ds(base0 + j * C1, C1)],
                         wsem[b])

    def wwait(b):
        pltpu.make_async_copy(
            rbuf[b], rbf_hbm.at[pl.ds(0, C1)], wsem[b]).wait()

    nchunk = EPW // C1
    compute_chunk(0, 0)
    wstart(0, 0)
    compute_chunk(1, 1)
    wstart(1, 1)

    def pair_body(p, carry):
        for jj, b in ((2, 0), (3, 1)):
            j = 2 * p + jj
            wwait(b)
            compute_chunk(j, b)
            wstart(j, b)
        return carry

    lax.fori_loop(0, (nchunk - 3) // 2, pair_body, 0)

    wwait(0)
    compute_chunk(nchunk - 1, 0)
    wstart(nchunk - 1, 0)
    wwait(0)
    wwait(1)


_NCHUNK = EPW // C2


def _sc2_body(src_hbm, dst_hbm, qi_hbm, h_hbm, t_hbm, agg0_hbm, agg1_hbm,
              agg_sh, src_all, qi3, dst3, hs3, ew3,
              g0, g1, g2, e0, e1, e2, q0, q1, q2, d0, d1, d2, s0, s1, s2):
    c = lax.axis_index("c")
    s = lax.axis_index("s")
    gsem = (g0, g1, g2)
    esem = (e0, e1, e2)
    qsem = (q0, q1, q2)
    dsem = (d0, d1, d2)
    ssem = (s0, s1, s2)

    def zrow(r, carry):
        for j in range(HID // 16):
            hs3[0, r, pl.ds(j * 16, 16)] = jnp.zeros((16,), jnp.float32)
        return carry

    lax.fori_loop(0, C2, zrow, 0)
    row0 = s * ROWS_PER_TILE
    for q in range(ROWS_PER_TILE // C2):
        pltpu.sync_copy(hs3.at[0], agg_sh.at[pl.ds(row0 + q * C2, C2)])
    plsc.subcore_barrier()

    base0 = (c * NS + s) * EPW
    pltpu.sync_copy(src_hbm.at[pl.ds(base0, EPW)], src_all)

    def fetch_a(j, b):
        base = base0 + j * C2
        loc = j * C2
        pltpu.async_copy(dst_hbm.at[pl.ds(base, C2)], dst3.at[b], dsem[b])
        pltpu.async_copy(qi_hbm.at[pl.ds(base, C2)], qi3.at[b], qsem[b])
        pltpu.async_copy(h_hbm.at[src_all.at[pl.ds(loc, C2)]], hs3.at[b],
                         gsem[b])

    def fetch_b(b):
        pltpu.make_async_copy(qi_hbm.at[pl.ds(0, C2)], qi3.at[b],
                              qsem[b]).wait()
        pltpu.async_copy(t_hbm.at[qi3.at[b]], ew3.at[b], esem[b])

    def wait_in(b):
        pltpu.make_async_copy(h_hbm.at[src_all.at[pl.ds(0, C2)]], hs3.at[b],
                              gsem[b]).wait()
        pltpu.make_async_copy(t_hbm.at[qi3.at[b]], ew3.at[b],
                              esem[b]).wait()
        pltpu.make_async_copy(dst_hbm.at[pl.ds(0, C2)], dst3.at[b],
                              dsem[b]).wait()

    def mult(b):
        def mrow(r, carry2):
            for j in range(HID // 16):
                sl = pl.ds(j * 16, 16)
                hs3[b, r, sl] = hs3[b, r, sl] * ew3[b, r, sl]
            return carry2

        lax.fori_loop(0, C2, mrow, 0)

    def scat_start(b):
        pltpu.async_copy(hs3.at[b], agg_sh.at[dst3.at[b]], ssem[b], add=True)

    def scat_wait(b):
        pltpu.make_async_copy(
            hs3.at[b], agg_sh.at[dst3.at[b]], ssem[b]).wait()

    fetch_a(0, 0)
    fetch_a(1, 1)
    fetch_b(0)
    wait_in(0)
    mult(0)
    scat_start(0)
    fetch_a(2, 2)
    fetch_b(1)

    def tri_body(p, carry):
        for jj, b in ((1, 1), (2, 2), (3, 0)):
            j = 3 * p + jj
            wait_in(b)
            mult(b)
            scat_start(b)
            b2 = (b + 2) % 3
            scat_wait(b2)
            fetch_a(j + 2, b2)
            fetch_b((b + 1) % 3)
        return carry

    lax.fori_loop(0, (_NCHUNK - 4) // 3, tri_body, 0)

    wait_in(1)
    mult(1)
    scat_start(1)
    scat_wait(0)
    fetch_a(_NCHUNK - 1, 0)
    fetch_b(2)
    wait_in(2)
    mult(2)
    scat_start(2)
    fetch_b(0)
    wait_in(0)
    mult(0)
    scat_start(0)
    scat_wait(1)
    scat_wait(2)
    scat_wait(0)
    plsc.subcore_barrier()

    rows = pl.ds(row0, ROWS_PER_TILE)

    @pl.when(c == 0)
    def _():
        pltpu.sync_copy(agg_sh.at[rows], agg0_hbm.at[rows])

    @pl.when(c == 1)
    def _():
        pltpu.sync_copy(agg_sh.at[rows], agg1_hbm.at[rows])


def _tca_body(an_ref, emb_ref, h_ref):
    an = an_ref[...]
    col = lax.broadcasted_iota(jnp.int32, (1, NUM_EMB), 1)
    oh = (an == col).astype(jnp.float32)
    h_ref[...] = jnp.dot(oh, emb_ref[...], preferred_element_type=jnp.float32)


_QB = 1024


def _tcb_body(wr_ref, t_ref):
    i = pl.program_id(0)
    q = lax.broadcasted_iota(jnp.int32, (_QB, NUM_RBF), 0) + i * _QB
    d = q.astype(jnp.float32) * DELTA
    ck = lax.broadcasted_iota(
        jnp.int32, (_QB, NUM_RBF), 1).astype(jnp.float32) * RBF_STEP
    t = d - ck
    rbf = jnp.exp(-GAMMA * (t * t))
    t_ref[...] = jnp.dot(rbf, wr_ref[...], preferred_element_type=jnp.float32)


def _tcc_body(an_ref, h_ref, a0_ref, a1_ref, ws_ref, wn_ref,
              out_ref, vn_ref, pn_ref):
    h = h_ref[...]
    agg = a0_ref[...] + a1_ref[...]
    z = (jnp.dot(h, ws_ref[...], preferred_element_type=jnp.float32)
         + jnp.dot(agg, wn_ref[...], preferred_element_type=jnp.float32))
    out = z * jax.nn.sigmoid(z)
    vn = (an_ref[...] == VNODE_Z).astype(jnp.float32)
    out_ref[...] = out
    vn_ref[...] = out * vn
    pn_ref[...] = out * (1.0 - vn)


@functools.cache
def _sc_kernels():
    mesh = plsc.VectorSubcoreMesh(
        core_axis_name="c", subcore_axis_name="s",
        num_cores=NC, num_subcores=NS)
    sc_params = pltpu.CompilerParams(needs_layout_passes=False)
    sc1 = pl.kernel(
        _sc1_body,
        out_type=(jax.ShapeDtypeStruct((N_EDGES,), jnp.int32),
                  jax.ShapeDtypeStruct((N_NODES, HID), jnp.float32)),
        mesh=mesh,
        compiler_params=sc_params,
        scratch_types=[
            pltpu.VMEM((N_NODES * 3,), jnp.float32),
            pltpu.VMEM((EPW,), jnp.int32),
            pltpu.VMEM((EPW,), jnp.int32),
            pltpu.VMEM((C1,), jnp.int32),
            pltpu.VMEM((C1,), jnp.int32),
            pltpu.VMEM((_HCH,), jnp.int32),
            pltpu.VMEM((_HCH, HID), jnp.float32),
            pltpu.SemaphoreType.DMA,
            pltpu.SemaphoreType.DMA,
            pltpu.SemaphoreType.DMA,
        ],
    )
    sc2 = pl.kernel(
        _sc2_body,
        out_type=(jax.ShapeDtypeStruct((N_PAD, HID), jnp.float32),
                  jax.ShapeDtypeStruct((N_PAD, HID), jnp.float32)),
        mesh=mesh,
        compiler_params=sc_params,
        scratch_types=[
            pltpu.VMEM_SHARED((N_PAD, HID), jnp.float32),
            pltpu.VMEM((EPW,), jnp.int32),
            pltpu.VMEM((3, C2), jnp.int32),
            pltpu.VMEM((3, C2), jnp.int32),
            pltpu.VMEM((3, C2, HID), jnp.float32),
            pltpu.VMEM((3, C2, HID), jnp.float32),
        ] + [pltpu.SemaphoreType.DMA] * 15,
    )
    return sc1, sc2

_NB = 1000
_EB = 2560


@jax.jit
def _tca(an2, emb):
    return pl.pallas_call(
        _tca_body,
        grid=(N_NODES // _NB,),
        in_specs=[
            pl.BlockSpec((_NB, 1), lambda i: (i, 0)),
            pl.BlockSpec((NUM_EMB, HID), lambda i: (0, 0)),
        ],
        out_specs=pl.BlockSpec((_NB, HID), lambda i: (i, 0)),
        out_shape=jax.ShapeDtypeStruct((N_NODES, HID), jnp.float32),
    )(an2, emb)


@jax.jit
def _tcb(W_rbf):
    return pl.pallas_call(
        _tcb_body,
        grid=(QLUT // _QB,),
        in_specs=[
            pl.BlockSpec((NUM_RBF, HID), lambda i: (0, 0)),
        ],
        out_specs=pl.BlockSpec((_QB, HID), lambda i: (i, 0)),
        out_shape=jax.ShapeDtypeStruct((QLUT, HID), jnp.float32),
    )(W_rbf)


@jax.jit
def _tcc(an2, h, agg0, agg1, W_self, W_nbr):
    return pl.pallas_call(
        _tcc_body,
        grid=(N_NODES // _NB,),
        in_specs=[
            pl.BlockSpec((_NB, 1), lambda i: (i, 0)),
            pl.BlockSpec((_NB, HID), lambda i: (i, 0)),
            pl.BlockSpec((_NB, HID), lambda i: (i, 0)),
            pl.BlockSpec((_NB, HID), lambda i: (i, 0)),
            pl.BlockSpec((HID, HID), lambda i: (0, 0)),
            pl.BlockSpec((HID, HID), lambda i: (0, 0)),
        ],
        out_specs=[
            pl.BlockSpec((_NB, HID), lambda i: (i, 0)),
            pl.BlockSpec((_NB, HID), lambda i: (i, 0)),
            pl.BlockSpec((_NB, HID), lambda i: (i, 0)),
        ],
        out_shape=(jax.ShapeDtypeStruct((N_NODES, HID), jnp.float32),
                   jax.ShapeDtypeStruct((N_NODES, HID), jnp.float32),
                   jax.ShapeDtypeStruct((N_NODES, HID), jnp.float32)),
    )(an2, h, agg0, agg1, W_self, W_nbr)


def kernel(atomic_numbers, edge_index, pos, emb, W_self, W_nbr, W_rbf):
    src = edge_index[0]
    dst = edge_index[1]
    an2 = atomic_numbers.reshape(N_NODES, 1)

    sc1, sc2 = _sc_kernels()
    qi, h = sc1(src, dst, pos.reshape(-1), atomic_numbers, emb)
    t_lut = _tcb(W_rbf)
    agg0, agg1 = sc2(src, dst, qi, h, t_lut)
    out, vn_feat, pn_feat = _tcc(an2, h, agg0, agg1, W_self, W_nbr)
    return (out, vn_feat, pn_feat)


# speedup vs baseline: 18.0159x; 18.0159x over previous
import functools

import jax
import jax.numpy as jnp
from jax import lax
from jax.experimental import pallas as pl
from jax.experimental.pallas import tpu as pltpu
from jax.experimental.pallas import tpu_sc as plsc

N_NODES = 10000
N_EDGES = 320000
HID = 128
NUM_EMB = 125
NUM_RBF = 16
CUTOFF = 12.0
VNODE_Z = 124
GAMMA = (CUTOFF / NUM_RBF) ** -2
RBF_STEP = CUTOFF / (NUM_RBF - 1)

NC = 2
NS = 16
NW = NC * NS
EPW = N_EDGES // NW
N_PAD = 10240
ROWS_PER_TILE = N_PAD // NS

C1 = 400
C2 = 40
ZR = 128


_HCH = 80
_NHCH = N_NODES // _HCH


def _sc1_body(src_hbm, dst_hbm, pos_hbm, an_hbm, emb_hbm, rbf_hbm, h_hbm,
              pos_v, src_all, dst_all, rb0, rb1, an_v, h_v,
              hsem, w0, w1):
    c = lax.axis_index("c")
    s = lax.axis_index("s")
    wid = c * NS + s
    base0 = wid * EPW
    pltpu.sync_copy(pos_hbm, pos_v)

    nch = 3 + (wid < _NHCH - 3 * NW).astype(jnp.int32)

    def hbody(q, carry):
        base = (wid + q * NW) * _HCH
        pltpu.sync_copy(an_hbm.at[pl.ds(base, _HCH)], an_v)
        pltpu.async_copy(emb_hbm.at[an_v], h_v, hsem).wait()
        pltpu.sync_copy(h_v, h_hbm.at[pl.ds(base, _HCH)])
        return carry

    lax.fori_loop(0, nch, hbody, 0)

    pltpu.sync_copy(src_hbm.at[pl.ds(base0, EPW)], src_all)
    pltpu.sync_copy(dst_hbm.at[pl.ds(base0, EPW)], dst_all)

    lane16 = lax.broadcasted_iota(jnp.int32, (16,), 0) * NUM_RBF
    rbuf = (rb0, rb1)
    wsem = (w0, w1)

    def compute_chunk(j, b):
        def vec_body(k, carry2):
            loc = pl.multiple_of(j * C1 + k * 16, 16)
            si = src_all[pl.ds(loc, 16)] * 3
            di = dst_all[pl.ds(loc, 16)] * 3
            sx = plsc.load_gather(pos_v, [si])
            sy = plsc.load_gather(pos_v, [si + 1])
            sz = plsc.load_gather(pos_v, [si + 2])
            dx = plsc.load_gather(pos_v, [di]) - sx
            dy = plsc.load_gather(pos_v, [di + 1]) - sy
            dz = plsc.load_gather(pos_v, [di + 2]) - sz
            x = dx * dx + dy * dy + dz * dz + 1e-12
            ii = plsc.bitcast(x, jnp.int32)
            ii = 0x5F3759DF - lax.shift_right_logical(ii, 1)
            y = plsc.bitcast(ii, jnp.float32)
            for _ in range(4):
                y = y * (1.5 - 0.5 * x * y * y)
            d = x * y
            rb = lane16 + (k * 16) * NUM_RBF
            for kk in range(NUM_RBF):
                t = d - (kk * RBF_STEP)
                plsc.store_scatter(rbuf[b], [rb + kk],
                                   jnp.exp(-GAMMA * (t * t)))
            return carry2

        lax.fori_loop(0, C1 // 16, vec_body, 0)

    def wstart(j, b):
        pltpu.async_copy(
            rbuf[b],
            rbf_hbm.at[pl.ds((base0 + j * C1) * NUM_RBF, C1 * NUM_RBF)],
            wsem[b])

    def wwait(b):
        pltpu.make_async_copy(
            rbuf[b], rbf_hbm.at[pl.ds(0, C1 * NUM_RBF)], wsem[b]).wait()

    nchunk = EPW // C1
    compute_chunk(0, 0)
    wstart(0, 0)
    compute_chunk(1, 1)
    wstart(1, 1)

    def pair_body(p, carry):
        for jj, b in ((2, 0), (3, 1)):
            j = 2 * p + jj
            wwait(b)
            compute_chunk(j, b)
            wstart(j, b)
        return carry

    lax.fori_loop(0, (nchunk - 3) // 2, pair_body, 0)

    wwait(0)
    compute_chunk(nchunk - 1, 0)
    wstart(nchunk - 1, 0)
    wwait(0)
    wwait(1)


_NCHUNK = EPW // C2


def _sc2_body(src_hbm, dst_hbm, h_hbm, ew_hbm, agg0_hbm, agg1_hbm,
              agg_sh, src_all, dst3, hs3, ew3,
              g0, g1, g2, e0, e1, e2, d0, d1, d2, s0, s1, s2):
    c = lax.axis_index("c")
    s = lax.axis_index("s")
    gsem = (g0, g1, g2)
    esem = (e0, e1, e2)
    dsem = (d0, d1, d2)
    ssem = (s0, s1, s2)

    def zrow(r, carry):
        for j in range(HID // 16):
            hs3[0, r, pl.ds(j * 16, 16)] = jnp.zeros((16,), jnp.float32)
        return carry

    lax.fori_loop(0, C2, zrow, 0)
    row0 = s * ROWS_PER_TILE
    for q in range(ROWS_PER_TILE // C2):
        pltpu.sync_copy(hs3.at[0], agg_sh.at[pl.ds(row0 + q * C2, C2)])
    plsc.subcore_barrier()

    base0 = (c * NS + s) * EPW
    pltpu.sync_copy(src_hbm.at[pl.ds(base0, EPW)], src_all)

    def fetch(j, b):
        base = base0 + j * C2
        loc = j * C2
        pltpu.async_copy(dst_hbm.at[pl.ds(base, C2)], dst3.at[b], dsem[b])
        pltpu.async_copy(h_hbm.at[src_all.at[pl.ds(loc, C2)]], hs3.at[b],
                         gsem[b])
        pltpu.async_copy(ew_hbm.at[pl.ds(base, C2)], ew3.at[b], esem[b])

    def wait_in(b):
        pltpu.make_async_copy(h_hbm.at[src_all.at[pl.ds(0, C2)]], hs3.at[b],
                              gsem[b]).wait()
        pltpu.make_async_copy(ew_hbm.at[pl.ds(0, C2)], ew3.at[b],
                              esem[b]).wait()
        pltpu.make_async_copy(dst_hbm.at[pl.ds(0, C2)], dst3.at[b],
                              dsem[b]).wait()

    def mult(b):
        @plsc.parallel_loop(0, C2, step=1, unroll=4)
        def mrow(r):
            for j in range(HID // 16):
                sl = pl.ds(j * 16, 16)
                hs3[b, r, sl] = hs3[b, r, sl] * ew3[b, r, sl]

    def scat_start(b):
        pltpu.async_copy(hs3.at[b], agg_sh.at[dst3.at[b]], ssem[b], add=True)

    def scat_wait(b):
        pltpu.make_async_copy(
            hs3.at[b], agg_sh.at[dst3.at[b]], ssem[b]).wait()

    fetch(0, 0)
    fetch(1, 1)
    wait_in(0)
    mult(0)
    scat_start(0)
    fetch(2, 2)

    def tri_body(p, carry):
        for jj, b in ((1, 1), (2, 2), (3, 0)):
            j = 3 * p + jj
            wait_in(b)
            mult(b)
            scat_start(b)
            o = (b + 2) % 3
            scat_wait(o)
            fetch(j + 2, o)
        return carry

    lax.fori_loop(0, (_NCHUNK - 4) // 3, tri_body, 0)

    wait_in(1)
    mult(1)
    scat_start(1)
    scat_wait(0)
    fetch(_NCHUNK - 1, 0)
    wait_in(2)
    mult(2)
    scat_start(2)
    wait_in(0)
    mult(0)
    scat_start(0)
    scat_wait(1)
    scat_wait(2)
    scat_wait(0)
    plsc.subcore_barrier()

    rows = pl.ds(row0, ROWS_PER_TILE)

    @pl.when(c == 0)
    def _():
        pltpu.sync_copy(agg_sh.at[rows], agg0_hbm.at[rows])

    @pl.when(c == 1)
    def _():
        pltpu.sync_copy(agg_sh.at[rows], agg1_hbm.at[rows])


def _tca_body(an_ref, emb_ref, h_ref):
    an = an_ref[...]
    col = lax.broadcasted_iota(jnp.int32, (1, NUM_EMB), 1)
    oh = (an == col).astype(jnp.float32)
    h_ref[...] = jnp.dot(oh, emb_ref[...], preferred_element_type=jnp.float32)


_RB = 160


def _tcb_body(r_ref, w_ref, ew_ref):
    rbf2 = r_ref[...]
    a = jnp.broadcast_to(rbf2[:, None, :], (_RB, 8, 128)).reshape(_RB * 8, 128)
    rowp = lax.broadcasted_iota(jnp.int32, (_RB * 8, 128), 0) % 8
    colg = lax.broadcasted_iota(jnp.int32, (_RB * 8, 128), 1) // NUM_RBF
    am = jnp.where(rowp == colg, a, 0.0)
    ew_ref[...] = jnp.dot(am, w_ref[...], preferred_element_type=jnp.float32)


def _tcc_body(an_ref, h_ref, a0_ref, a1_ref, ws_ref, wn_ref,
              out_ref, vn_ref, pn_ref):
    h = h_ref[...]
    agg = a0_ref[...] + a1_ref[...]
    z = (jnp.dot(h, ws_ref[...], preferred_element_type=jnp.float32)
         + jnp.dot(agg, wn_ref[...], preferred_element_type=jnp.float32))
    out = z * jax.nn.sigmoid(z)
    vn = (an_ref[...] == VNODE_Z).astype(jnp.float32)
    out_ref[...] = out
    vn_ref[...] = out * vn
    pn_ref[...] = out * (1.0 - vn)


@functools.cache
def _sc_kernels():
    mesh = plsc.VectorSubcoreMesh(
        core_axis_name="c", subcore_axis_name="s",
        num_cores=NC, num_subcores=NS)
    sc_params = pltpu.CompilerParams(needs_layout_passes=False)
    sc1 = pl.kernel(
        _sc1_body,
        out_type=(jax.ShapeDtypeStruct((N_EDGES * NUM_RBF,), jnp.float32),
                  jax.ShapeDtypeStruct((N_NODES, HID), jnp.float32)),
        mesh=mesh,
        compiler_params=sc_params,
        scratch_types=[
            pltpu.VMEM((N_NODES * 3,), jnp.float32),
            pltpu.VMEM((EPW,), jnp.int32),
            pltpu.VMEM((EPW,), jnp.int32),
            pltpu.VMEM((C1 * NUM_RBF,), jnp.float32),
            pltpu.VMEM((C1 * NUM_RBF,), jnp.float32),
            pltpu.VMEM((_HCH,), jnp.int32),
            pltpu.VMEM((_HCH, HID), jnp.float32),
            pltpu.SemaphoreType.DMA,
            pltpu.SemaphoreType.DMA,
            pltpu.SemaphoreType.DMA,
        ],
    )
    sc2 = pl.kernel(
        _sc2_body,
        out_type=(jax.ShapeDtypeStruct((N_PAD, HID), jnp.float32),
                  jax.ShapeDtypeStruct((N_PAD, HID), jnp.float32)),
        mesh=mesh,
        compiler_params=sc_params,
        scratch_types=[
            pltpu.VMEM_SHARED((N_PAD, HID), jnp.float32),
            pltpu.VMEM((EPW,), jnp.int32),
            pltpu.VMEM((3, C2), jnp.int32),
            pltpu.VMEM((3, C2, HID), jnp.float32),
            pltpu.VMEM((3, C2, HID), jnp.float32),
        ] + [pltpu.SemaphoreType.DMA] * 12,
    )
    return sc1, sc2

_NB = 1000
_EB = 2560


@jax.jit
def _tca(an2, emb):
    return pl.pallas_call(
        _tca_body,
        grid=(N_NODES // _NB,),
        in_specs=[
            pl.BlockSpec((_NB, 1), lambda i: (i, 0)),
            pl.BlockSpec((NUM_EMB, HID), lambda i: (0, 0)),
        ],
        out_specs=pl.BlockSpec((_NB, HID), lambda i: (i, 0)),
        out_shape=jax.ShapeDtypeStruct((N_NODES, HID), jnp.float32),
    )(an2, emb)


@jax.jit
def _tcb(rbf2, W16):
    nrows = N_EDGES * NUM_RBF // HID
    return pl.pallas_call(
        _tcb_body,
        grid=(nrows // _RB,),
        in_specs=[
            pl.BlockSpec((_RB, HID), lambda i: (i, 0)),
            pl.BlockSpec((HID, HID), lambda i: (0, 0)),
        ],
        out_specs=pl.BlockSpec((_RB * 8, HID), lambda i: (i, 0)),
        out_shape=jax.ShapeDtypeStruct((N_EDGES, HID), jnp.float32),
    )(rbf2, W16)


@jax.jit
def _tcc(an2, h, agg0, agg1, W_self, W_nbr):
    return pl.pallas_call(
        _tcc_body,
        grid=(N_NODES // _NB,),
        in_specs=[
            pl.BlockSpec((_NB, 1), lambda i: (i, 0)),
            pl.BlockSpec((_NB, HID), lambda i: (i, 0)),
            pl.BlockSpec((_NB, HID), lambda i: (i, 0)),
            pl.BlockSpec((_NB, HID), lambda i: (i, 0)),
            pl.BlockSpec((HID, HID), lambda i: (0, 0)),
            pl.BlockSpec((HID, HID), lambda i: (0, 0)),
        ],
        out_specs=[
            pl.BlockSpec((_NB, HID), lambda i: (i, 0)),
            pl.BlockSpec((_NB, HID), lambda i: (i, 0)),
            pl.BlockSpec((_NB, HID), lambda i: (i, 0)),
        ],
        out_shape=(jax.ShapeDtypeStruct((N_NODES, HID), jnp.float32),
                   jax.ShapeDtypeStruct((N_NODES, HID), jnp.float32),
                   jax.ShapeDtypeStruct((N_NODES, HID), jnp.float32)),
    )(an2, h, agg0, agg1, W_self, W_nbr)


def kernel(atomic_numbers, edge_index, pos, emb, W_self, W_nbr, W_rbf):
    src = edge_index[0]
    dst = edge_index[1]
    an2 = atomic_numbers.reshape(N_NODES, 1)

    sc1, sc2 = _sc_kernels()
    rbf_flat, h = sc1(src, dst, pos.reshape(-1), atomic_numbers, emb)
    rbf2 = rbf_flat.reshape(N_EDGES * NUM_RBF // HID, HID)
    W16 = jnp.tile(W_rbf, (8, 1))
    ew = _tcb(rbf2, W16)
    agg0, agg1 = sc2(src, dst, h, ew)
    out, vn_feat, pn_feat = _tcc(an2, h, agg0, agg1, W_self, W_nbr)
    return (out, vn_feat, pn_feat)


# speedup vs baseline: 18.0966x; 1.0045x over previous
import functools

import jax
import jax.numpy as jnp
from jax import lax
from jax.experimental import pallas as pl
from jax.experimental.pallas import tpu as pltpu
from jax.experimental.pallas import tpu_sc as plsc

N_NODES = 10000
N_EDGES = 320000
HID = 128
NUM_EMB = 125
NUM_RBF = 16
CUTOFF = 12.0
VNODE_Z = 124
GAMMA = (CUTOFF / NUM_RBF) ** -2
RBF_STEP = CUTOFF / (NUM_RBF - 1)

NC = 2
NS = 16
NW = NC * NS
EPW = N_EDGES // NW
N_PAD = 10240
ROWS_PER_TILE = N_PAD // NS

C1 = 400
C2 = 40
ZR = 128


_HCH = 80
_NHCH = N_NODES // _HCH


def _sc1_body(src_hbm, dst_hbm, pos_hbm, an_hbm, emb_hbm, rbf_hbm, h_hbm,
              pos_v, src_all, dst_all, rb0, rb1, an_v, h_v,
              hsem, w0, w1):
    c = lax.axis_index("c")
    s = lax.axis_index("s")
    wid = c * NS + s
    base0 = wid * EPW
    pltpu.sync_copy(pos_hbm, pos_v)

    nch = 3 + (wid < _NHCH - 3 * NW).astype(jnp.int32)

    def hbody(q, carry):
        base = (wid + q * NW) * _HCH
        pltpu.sync_copy(an_hbm.at[pl.ds(base, _HCH)], an_v)
        pltpu.async_copy(emb_hbm.at[an_v], h_v, hsem).wait()
        pltpu.sync_copy(h_v, h_hbm.at[pl.ds(base, _HCH)])
        return carry

    lax.fori_loop(0, nch, hbody, 0)

    pltpu.sync_copy(src_hbm.at[pl.ds(base0, EPW)], src_all)
    pltpu.sync_copy(dst_hbm.at[pl.ds(base0, EPW)], dst_all)

    lane16 = lax.broadcasted_iota(jnp.int32, (16,), 0) * NUM_RBF
    rbuf = (rb0, rb1)
    wsem = (w0, w1)

    def compute_chunk(j, b):
        def vec_body(k, carry2):
            loc = pl.multiple_of(j * C1 + k * 16, 16)
            si = src_all[pl.ds(loc, 16)] * 3
            di = dst_all[pl.ds(loc, 16)] * 3
            sx = plsc.load_gather(pos_v, [si])
            sy = plsc.load_gather(pos_v, [si + 1])
            sz = plsc.load_gather(pos_v, [si + 2])
            dx = plsc.load_gather(pos_v, [di]) - sx
            dy = plsc.load_gather(pos_v, [di + 1]) - sy
            dz = plsc.load_gather(pos_v, [di + 2]) - sz
            x = dx * dx + dy * dy + dz * dz + 1e-12
            ii = plsc.bitcast(x, jnp.int32)
            ii = 0x5F3759DF - lax.shift_right_logical(ii, 1)
            y = plsc.bitcast(ii, jnp.float32)
            for _ in range(4):
                y = y * (1.5 - 0.5 * x * y * y)
            d = x * y
            rb = lane16 + (k * 16) * NUM_RBF
            for kk in range(NUM_RBF):
                t = d - (kk * RBF_STEP)
                plsc.store_scatter(rbuf[b], [rb + kk],
                                   jnp.exp(-GAMMA * (t * t)))
            return carry2

        lax.fori_loop(0, C1 // 16, vec_body, 0)

    def wstart(j, b):
        pltpu.async_copy(
            rbuf[b],
            rbf_hbm.at[pl.ds((base0 + j * C1) * NUM_RBF, C1 * NUM_RBF)],
            wsem[b])

    def wwait(b):
        pltpu.make_async_copy(
            rbuf[b], rbf_hbm.at[pl.ds(0, C1 * NUM_RBF)], wsem[b]).wait()

    nchunk = EPW // C1
    compute_chunk(0, 0)
    wstart(0, 0)
    compute_chunk(1, 1)
    wstart(1, 1)

    def pair_body(p, carry):
        for jj, b in ((2, 0), (3, 1)):
            j = 2 * p + jj
            wwait(b)
            compute_chunk(j, b)
            wstart(j, b)
        return carry

    lax.fori_loop(0, (nchunk - 3) // 2, pair_body, 0)

    wwait(0)
    compute_chunk(nchunk - 1, 0)
    wstart(nchunk - 1, 0)
    wwait(0)
    wwait(1)


_NCHUNK = EPW // C2


def _sc2_body(src_hbm, dst_hbm, h_hbm, ew_hbm, agg0_hbm, agg1_hbm,
              agg_sh, src_all, dst3, hs3, ew3,
              g0, g1, g2, e0, e1, e2, d0, d1, d2, s0, s1, s2):
    c = lax.axis_index("c")
    s = lax.axis_index("s")
    gsem = (g0, g1, g2)
    esem = (e0, e1, e2)
    dsem = (d0, d1, d2)
    ssem = (s0, s1, s2)

    def zrow(r, carry):
        for j in range(HID // 16):
            hs3[0, r, pl.ds(j * 16, 16)] = jnp.zeros((16,), jnp.float32)
        return carry

    lax.fori_loop(0, C2, zrow, 0)
    row0 = s * ROWS_PER_TILE
    for q in range(ROWS_PER_TILE // C2):
        pltpu.sync_copy(hs3.at[0], agg_sh.at[pl.ds(row0 + q * C2, C2)])
    plsc.subcore_barrier()

    base0 = (c * NS + s) * EPW
    pltpu.sync_copy(src_hbm.at[pl.ds(base0, EPW)], src_all)

    def fetch(j, b):
        base = base0 + j * C2
        loc = j * C2
        pltpu.async_copy(dst_hbm.at[pl.ds(base, C2)], dst3.at[b], dsem[b])
        pltpu.async_copy(h_hbm.at[src_all.at[pl.ds(loc, C2)]], hs3.at[b],
                         gsem[b])
        pltpu.async_copy(ew_hbm.at[pl.ds(base, C2)], ew3.at[b], esem[b])

    def wait_in(b):
        pltpu.make_async_copy(h_hbm.at[src_all.at[pl.ds(0, C2)]], hs3.at[b],
                              gsem[b]).wait()
        pltpu.make_async_copy(ew_hbm.at[pl.ds(0, C2)], ew3.at[b],
                              esem[b]).wait()
        pltpu.make_async_copy(dst_hbm.at[pl.ds(0, C2)], dst3.at[b],
                              dsem[b]).wait()

    def mult(b):
        def mrow(r, carry2):
            for j in range(HID // 16):
                sl = pl.ds(j * 16, 16)
                hs3[b, r, sl] = hs3[b, r, sl] * ew3[b, r, sl]
            return carry2

        lax.fori_loop(0, C2, mrow, 0)

    def scat_start(b):
        pltpu.async_copy(hs3.at[b], agg_sh.at[dst3.at[b]], ssem[b], add=True)

    def scat_wait(b):
        pltpu.make_async_copy(
            hs3.at[b], agg_sh.at[dst3.at[b]], ssem[b]).wait()

    fetch(0, 0)
    fetch(1, 1)
    wait_in(0)
    mult(0)
    scat_start(0)
    fetch(2, 2)

    def tri_body(p, carry):
        for jj, b in ((1, 1), (2, 2), (3, 0)):
            j = 3 * p + jj
            wait_in(b)
            mult(b)
            scat_start(b)
            o = (b + 2) % 3
            scat_wait(o)
            fetch(j + 2, o)
        return carry

    lax.fori_loop(0, (_NCHUNK - 4) // 3, tri_body, 0)

    wait_in(1)
    mult(1)
    scat_start(1)
    scat_wait(0)
    fetch(_NCHUNK - 1, 0)
    wait_in(2)
    mult(2)
    scat_start(2)
    wait_in(0)
    mult(0)
    scat_start(0)
    scat_wait(1)
    scat_wait(2)
    scat_wait(0)
    plsc.subcore_barrier()

    rows = pl.ds(row0, ROWS_PER_TILE)

    @pl.when(c == 0)
    def _():
        pltpu.sync_copy(agg_sh.at[rows], agg0_hbm.at[rows])

    @pl.when(c == 1)
    def _():
        pltpu.sync_copy(agg_sh.at[rows], agg1_hbm.at[rows])


def _tca_body(an_ref, emb_ref, h_ref):
    an = an_ref[...]
    col = lax.broadcasted_iota(jnp.int32, (1, NUM_EMB), 1)
    oh = (an == col).astype(jnp.float32)
    h_ref[...] = jnp.dot(oh, emb_ref[...], preferred_element_type=jnp.float32)


_RB = 160


def _tcb_body(r_ref, w_ref, ew_ref):
    rbf2 = r_ref[...]
    a = jnp.broadcast_to(rbf2[:, None, :], (_RB, 8, 128)).reshape(_RB * 8, 128)
    rowp = lax.broadcasted_iota(jnp.int32, (_RB * 8, 128), 0) % 8
    colg = lax.broadcasted_iota(jnp.int32, (_RB * 8, 128), 1) // NUM_RBF
    am = jnp.where(rowp == colg, a, 0.0)
    ew_ref[...] = jnp.dot(am, w_ref[...], preferred_element_type=jnp.float32)


def _tcc_body(an_ref, h_ref, a0_ref, a1_ref, ws_ref, wn_ref,
              out_ref, vn_ref, pn_ref):
    h = h_ref[...]
    agg = a0_ref[...] + a1_ref[...]
    z = (jnp.dot(h, ws_ref[...], preferred_element_type=jnp.float32)
         + jnp.dot(agg, wn_ref[...], preferred_element_type=jnp.float32))
    out = z * jax.nn.sigmoid(z)
    vn = (an_ref[...] == VNODE_Z).astype(jnp.float32)
    out_ref[...] = out
    vn_ref[...] = out * vn
    pn_ref[...] = out * (1.0 - vn)


@functools.cache
def _sc_kernels():
    mesh = plsc.VectorSubcoreMesh(
        core_axis_name="c", subcore_axis_name="s",
        num_cores=NC, num_subcores=NS)
    sc_params = pltpu.CompilerParams(needs_layout_passes=False)
    sc1 = pl.kernel(
        _sc1_body,
        out_type=(jax.ShapeDtypeStruct((N_EDGES * NUM_RBF,), jnp.float32),
                  jax.ShapeDtypeStruct((N_NODES, HID), jnp.float32)),
        mesh=mesh,
        compiler_params=sc_params,
        scratch_types=[
            pltpu.VMEM((N_NODES * 3,), jnp.float32),
            pltpu.VMEM((EPW,), jnp.int32),
            pltpu.VMEM((EPW,), jnp.int32),
            pltpu.VMEM((C1 * NUM_RBF,), jnp.float32),
            pltpu.VMEM((C1 * NUM_RBF,), jnp.float32),
            pltpu.VMEM((_HCH,), jnp.int32),
            pltpu.VMEM((_HCH, HID), jnp.float32),
            pltpu.SemaphoreType.DMA,
            pltpu.SemaphoreType.DMA,
            pltpu.SemaphoreType.DMA,
        ],
    )
    sc2 = pl.kernel(
        _sc2_body,
        out_type=(jax.ShapeDtypeStruct((N_PAD, HID), jnp.float32),
                  jax.ShapeDtypeStruct((N_PAD, HID), jnp.float32)),
        mesh=mesh,
        compiler_params=sc_params,
        scratch_types=[
            pltpu.VMEM_SHARED((N_PAD, HID), jnp.float32),
            pltpu.VMEM((EPW,), jnp.int32),
            pltpu.VMEM((3, C2), jnp.int32),
            pltpu.VMEM((3, C2, HID), jnp.float32),
            pltpu.VMEM((3, C2, HID), jnp.float32),
        ] + [pltpu.SemaphoreType.DMA] * 12,
    )
    return sc1, sc2

_NB = 1000
_EB = 2560


@jax.jit
def _tca(an2, emb):
    return pl.pallas_call(
        _tca_body,
        grid=(N_NODES // _NB,),
        in_specs=[
            pl.BlockSpec((_NB, 1), lambda i: (i, 0)),
            pl.BlockSpec((NUM_EMB, HID), lambda i: (0, 0)),
        ],
        out_specs=pl.BlockSpec((_NB, HID), lambda i: (i, 0)),
        out_shape=jax.ShapeDtypeStruct((N_NODES, HID), jnp.float32),
    )(an2, emb)


@jax.jit
def _tcb(rbf2, W16):
    nrows = N_EDGES * NUM_RBF // HID
    return pl.pallas_call(
        _tcb_body,
        grid=(nrows // _RB,),
        in_specs=[
            pl.BlockSpec((_RB, HID), lambda i: (i, 0)),
            pl.BlockSpec((HID, HID), lambda i: (0, 0)),
        ],
        out_specs=pl.BlockSpec((_RB * 8, HID), lambda i: (i, 0)),
        out_shape=jax.ShapeDtypeStruct((N_EDGES, HID), jnp.float32),
    )(rbf2, W16)


@jax.jit
def _tcc(an2, h, agg0, agg1, W_self, W_nbr):
    return pl.pallas_call(
        _tcc_body,
        grid=(N_NODES // _NB,),
        in_specs=[
            pl.BlockSpec((_NB, 1), lambda i: (i, 0)),
            pl.BlockSpec((_NB, HID), lambda i: (i, 0)),
            pl.BlockSpec((_NB, HID), lambda i: (i, 0)),
            pl.BlockSpec((_NB, HID), lambda i: (i, 0)),
            pl.BlockSpec((HID, HID), lambda i: (0, 0)),
            pl.BlockSpec((HID, HID), lambda i: (0, 0)),
        ],
        out_specs=[
            pl.BlockSpec((_NB, HID), lambda i: (i, 0)),
            pl.BlockSpec((_NB, HID), lambda i: (i, 0)),
            pl.BlockSpec((_NB, HID), lambda i: (i, 0)),
        ],
        out_shape=(jax.ShapeDtypeStruct((N_NODES, HID), jnp.float32),
                   jax.ShapeDtypeStruct((N_NODES, HID), jnp.float32),
                   jax.ShapeDtypeStruct((N_NODES, HID), jnp.float32)),
    )(an2, h, agg0, agg1, W_self, W_nbr)


def kernel(atomic_numbers, edge_index, pos, emb, W_self, W_nbr, W_rbf):
    src = edge_index[0]
    dst = edge_index[1]
    an2 = atomic_numbers.reshape(N_NODES, 1)

    sc1, sc2 = _sc_kernels()
    rbf_flat, h = sc1(src, dst, pos.reshape(-1), atomic_numbers, emb)
    rbf2 = rbf_flat.reshape(N_EDGES * NUM_RBF // HID, HID)
    W16 = jnp.tile(W_rbf, (8, 1))
    ew = _tcb(rbf2, W16)
    agg0, agg1 = sc2(src, dst, h, ew)
    out, vn_feat, pn_feat = _tcc(an2, h, agg0, agg1, W_self, W_nbr)
    return (out, vn_feat, pn_feat)
